# Initial kernel scaffold; baseline (speedup 1.0000x reference)
import jax
import jax.numpy as jnp
from jax.experimental import pallas as pl


def kernel(x, edge_attr, W1, b1, W2, b2, edge_index):
    raise NotImplementedError("write your pallas kernel here")


# speedup vs baseline: 13.1342x; 13.1342x over previous
import functools

import jax
import jax.numpy as jnp
from jax import lax
from jax.experimental import pallas as pl
from jax.experimental.pallas import tpu as pltpu
from jax.experimental.pallas import tpu_sc as plsc

N = 10000
E = 320000
D = 128
H = 128
C = 64

NC = 2
NS = 16
NW = NC * NS
CH = 128
NCH = 79
EPW = NCH * CH
PAD = EPW * NW - E
NPAD = 112
NA = N + NPAD
RPS = NA // NS

_mesh = plsc.VectorSubcoreMesh(core_axis_name="c", subcore_axis_name="s")


def _make_edge_prop(F):
    @functools.partial(
        pl.kernel,
        out_type=jax.ShapeDtypeStruct((NC, NA, F), jnp.float32),
        mesh=_mesh,
        scratch_types=[
            pltpu.VMEM((NCH, CH), jnp.int32),
            pltpu.VMEM((NCH, CH), jnp.int32),
            pltpu.VMEM((CH, F), jnp.float32),
            pltpu.VMEM_SHARED((NA, F), jnp.float32),
            pltpu.SemaphoreType.DMA,
        ],
    )
    def prop(y_hbm, src_hbm, dst_hbm, zero_hbm, out_hbm,
             src_v, dst_v, buf, acc, sem):
        c = lax.axis_index("c")
        s = lax.axis_index("s")
        w = s * NC + c
        pltpu.sync_copy(src_hbm.at[w], src_v)
        pltpu.sync_copy(dst_hbm.at[w], dst_v)
        r0 = s * RPS
        pltpu.sync_copy(zero_hbm.at[pl.ds(r0, RPS)], acc.at[pl.ds(r0, RPS)])
        plsc.subcore_barrier()

        def body(j, carry):
            pltpu.async_copy(y_hbm.at[src_v.at[j]], buf, sem).wait()
            pltpu.sync_copy(buf, acc.at[dst_v.at[j]], add=True)
            return carry

        lax.fori_loop(0, NCH, body, 0)
        plsc.subcore_barrier()
        pltpu.sync_copy(acc.at[pl.ds(r0, RPS)], out_hbm.at[c].at[pl.ds(r0, RPS)])

    return prop


_prop_h = _make_edge_prop(H)


@functools.partial(
    pl.kernel,
    out_type=jax.ShapeDtypeStruct((NC, NA, 16), jnp.float32),
    mesh=_mesh,
    scratch_types=[
        pltpu.VMEM((NCH, CH), jnp.int32),
        pltpu.VMEM((CH, 16), jnp.float32),
        pltpu.VMEM_SHARED((NA, 16), jnp.float32),
    ],
)
def _hist(dst_hbm, ones_hbm, zero_hbm, out_hbm, dst_v, ones_v, acc):
    c = lax.axis_index("c")
    s = lax.axis_index("s")
    w = s * NC + c
    pltpu.sync_copy(dst_hbm.at[w], dst_v)
    pltpu.sync_copy(ones_hbm, ones_v)
    r0 = s * RPS
    pltpu.sync_copy(zero_hbm.at[pl.ds(r0, RPS)], acc.at[pl.ds(r0, RPS)])
    plsc.subcore_barrier()

    def body(j, carry):
        pltpu.sync_copy(ones_v, acc.at[dst_v.at[j]], add=True)
        return carry

    lax.fori_loop(0, NCH, body, 0)
    plsc.subcore_barrier()
    pltpu.sync_copy(acc.at[pl.ds(r0, RPS)], out_hbm.at[c].at[pl.ds(r0, RPS)])


R = 1000


def _stage1_body(x_ref, w_ref, d0_ref, d1_ref, y_ref):
    deg = d0_ref[:, 0:1] + d1_ref[:, 0:1] + 1.0
    dinv = lax.rsqrt(deg)
    y_ref[...] = dinv * jnp.dot(x_ref[...], w_ref[...],
                                preferred_element_type=jnp.float32)


def _stage1(x, W1, deg0, deg1):
    return pl.pallas_call(
        _stage1_body,
        grid=(N // R,),
        in_specs=[
            pl.BlockSpec((R, D), lambda i: (i, 0)),
            pl.BlockSpec((D, H), lambda i: (0, 0)),
            pl.BlockSpec((R, 16), lambda i: (i, 0)),
            pl.BlockSpec((R, 16), lambda i: (i, 0)),
        ],
        out_specs=pl.BlockSpec((R, H), lambda i: (i, 0)),
        out_shape=jax.ShapeDtypeStruct((N, H), jnp.float32),
    )(x, W1, deg0, deg1)


def _stage2_body(p0_ref, p1_ref, y1_ref, d0_ref, d1_ref, b_ref, w_ref, o_ref):
    dinv = lax.rsqrt(d0_ref[:, 0:1] + d1_ref[:, 0:1] + 1.0)
    t = dinv * (p0_ref[...] + p1_ref[...] + y1_ref[...]) + b_ref[...]
    h = jnp.maximum(t, 0.0)
    y2 = dinv * jnp.dot(h, w_ref[...], preferred_element_type=jnp.float32)
    o_ref[...] = jnp.concatenate(
        [y2, jnp.zeros((y2.shape[0], H - C), jnp.float32)], axis=1)


def _stage2(p0, p1, y1, deg0, deg1, b1, W2):
    return pl.pallas_call(
        _stage2_body,
        grid=(N // R,),
        in_specs=[
            pl.BlockSpec((R, H), lambda i: (i, 0)),
            pl.BlockSpec((R, H), lambda i: (i, 0)),
            pl.BlockSpec((R, H), lambda i: (i, 0)),
            pl.BlockSpec((R, 16), lambda i: (i, 0)),
            pl.BlockSpec((R, 16), lambda i: (i, 0)),
            pl.BlockSpec((1, H), lambda i: (0, 0)),
            pl.BlockSpec((H, C), lambda i: (0, 0)),
        ],
        out_specs=pl.BlockSpec((R, H), lambda i: (i, 0)),
        out_shape=jax.ShapeDtypeStruct((N, H), jnp.float32),
    )(p0, p1, y1, deg0, deg1, b1, W2)


def _stage3_body(q0_ref, q1_ref, y2_ref, d0_ref, d1_ref, b_ref, o_ref):
    dinv = lax.rsqrt(d0_ref[:, 0:1] + d1_ref[:, 0:1] + 1.0)
    o = dinv * (q0_ref[:, :C] + q1_ref[:, :C] + y2_ref[:, :C]) + b_ref[...]
    m = jnp.max(o, axis=1, keepdims=True)
    lse = jnp.log(jnp.sum(jnp.exp(o - m), axis=1, keepdims=True)) + m
    o_ref[...] = o - lse


def _stage3(q0, q1, y2, deg0, deg1, b2):
    return pl.pallas_call(
        _stage3_body,
        grid=(N // R,),
        in_specs=[
            pl.BlockSpec((R, H), lambda i: (i, 0)),
            pl.BlockSpec((R, H), lambda i: (i, 0)),
            pl.BlockSpec((R, H), lambda i: (i, 0)),
            pl.BlockSpec((R, 16), lambda i: (i, 0)),
            pl.BlockSpec((R, 16), lambda i: (i, 0)),
            pl.BlockSpec((1, C), lambda i: (0, 0)),
        ],
        out_specs=pl.BlockSpec((R, C), lambda i: (i, 0)),
        out_shape=jax.ShapeDtypeStruct((N, C), jnp.float32),
    )(q0, q1, y2, deg0, deg1, b2)


def kernel(x, edge_attr, W1, b1, W2, b2, edge_index):
    src = edge_index[0]
    dst = edge_index[1]
    src_p = jnp.concatenate(
        [src, jnp.zeros((PAD,), jnp.int32)]).reshape(NW, NCH, CH)
    dst_p = jnp.concatenate(
        [dst, N + (jnp.arange(PAD, dtype=jnp.int32) % NPAD)]).reshape(NW, NCH, CH)

    ones16 = jnp.ones((CH, 16), jnp.float32)
    z16 = jnp.zeros((NA, 16), jnp.float32)
    zh = jnp.zeros((NA, H), jnp.float32)

    degp = _hist(dst_p, ones16, z16)
    deg0 = degp[0, :N, :]
    deg1 = degp[1, :N, :]

    y1 = _stage1(x, W1, deg0, deg1)
    p = _prop_h(y1, src_p, dst_p, zh)
    y2 = _stage2(p[0, :N], p[1, :N], y1, deg0, deg1,
                 b1.reshape(1, H), W2)
    q = _prop_h(y2, src_p, dst_p, zh)
    return _stage3(q[0, :N], q[1, :N], y2, deg0, deg1, b2.reshape(1, C))
